# Initial kernel scaffold; baseline (speedup 1.0000x reference)
import jax
import jax.numpy as jnp
from jax.experimental import pallas as pl


def kernel(voxel_features, voxel_coords, query_coords, key_indices, key_mask, W_qpos, b_qpos, W_kpos, b_kpos, W_in, b_in, W_ao, b_ao, W1, b1, W2, b2, g1, be1, W_o, b_o, g2, be2):
    raise NotImplementedError("write your pallas kernel here")


# speedup vs baseline: 3.1394x; 3.1394x over previous
import functools
import math

import jax
import jax.numpy as jnp
from jax import lax
from jax.experimental import pallas as pl
from jax.experimental.pallas import tpu as pltpu
from jax.experimental.pallas import tpu_sc as plsc

N1, N2, S, C, FF, H = 30000, 4096, 32, 256, 512, 8
DH = C // H
ROWS = N2 * S
B = 128
BS = B * S
NB = N2 // B
CP = 128

_NC, _NS = 2, 16
_NW = _NC * _NS
_RPW = ROWS // _NW
_CH = 128
_NCHUNK = _RPW // _CH


def _sc_gather_impl(feats, coords_p, idx_flat):
    mesh = plsc.VectorSubcoreMesh(core_axis_name="c", subcore_axis_name="s")

    @functools.partial(
        pl.kernel, mesh=mesh,
        out_type=[jax.ShapeDtypeStruct((ROWS, C), jnp.float32),
                  jax.ShapeDtypeStruct((ROWS, CP), jnp.float32)],
        scratch_types=[pltpu.VMEM((_CH,), jnp.int32),
                       pltpu.VMEM((_CH, C), jnp.float32),
                       pltpu.VMEM((_CH, CP), jnp.float32),
                       pltpu.SemaphoreType.DMA],
    )
    def gather_k(feat_hbm, coord_hbm, idx_hbm, gf_out, gc_out,
                 idx_v, fbuf, cbuf, sem):
        wid = lax.axis_index("s") * _NC + lax.axis_index("c")
        base = wid * _RPW

        def body(ci, carry):
            off = base + ci * _CH
            pltpu.sync_copy(idx_hbm.at[pl.ds(off, _CH)], idx_v)
            pltpu.async_copy(feat_hbm.at[idx_v], fbuf, sem).wait()
            pltpu.async_copy(coord_hbm.at[idx_v], cbuf, sem).wait()
            pltpu.sync_copy(fbuf, gf_out.at[pl.ds(off, _CH)])
            pltpu.sync_copy(cbuf, gc_out.at[pl.ds(off, _CH)])
            return carry

        lax.fori_loop(0, _NCHUNK, body, 0)

    return gather_k(feats, coords_p, idx_flat)


_sc_gather = _sc_gather_impl

_f32 = jnp.float32


def _dot(a, b, dims):
    return lax.dot_general(a, b, (dims, ((), ())),
                           preferred_element_type=_f32)


def _head_map():
    r = lax.broadcasted_iota(jnp.int32, (C, H), 0)
    c = lax.broadcasted_iota(jnp.int32, (C, H), 1)
    return ((r // DH) == c).astype(_f32)


def _main_body(gf_ref, gc_ref, qc_ref, mask_ref, wkp_ref, wqp_ref,
               win_ref, bin_ref, bkp_ref, bqp_ref, wao_ref, bao_ref,
               w1_ref, b1_ref, w2_ref, b2_ref,
               x1_ref, ssum_ref, gram_ref):
    i = pl.program_id(0)

    gf = gf_ref[...]
    gc = gc_ref[...]
    qc = qc_ref[...]

    rel = (gc.reshape(B, S, CP) - qc[:, None, :]).reshape(BS, CP)
    kpe = jax.nn.relu(_dot(rel, wkp_ref[...], ((1,), (1,))) + bkp_ref[...])
    k_in = gf + kpe

    win = win_ref[...]
    bin_ = bin_ref[...]
    k = _dot(k_in, win[C:2 * C, :], ((1,), (1,))) + bin_[:, C:2 * C]
    v = _dot(k_in, win[2 * C:, :], ((1,), (1,))) + bin_[:, 2 * C:]

    q_feat = jax.nn.relu(_dot(qc, wqp_ref[...], ((1,), (1,))) + bqp_ref[...])
    q = _dot(q_feat, win[:C, :], ((1,), (1,))) + bin_[:, :C]

    hm = _head_map()
    p = (k.reshape(B, S, C) * q[:, None, :]).reshape(BS, C)
    logits = _dot(p, hm, ((1,), (0,))) * (1.0 / math.sqrt(DH))
    l3 = logits.reshape(B, S, H) - mask_ref[...][:, :, None] * 1e30
    mx = jnp.max(l3, axis=1, keepdims=True)
    e = jnp.exp(l3 - mx)
    attn = e / jnp.sum(e, axis=1, keepdims=True)

    a_full = _dot(attn.reshape(BS, H), hm, ((1,), (1,)))
    o = jnp.sum((a_full * v).reshape(B, S, C), axis=1)

    attn_out = _dot(o, wao_ref[...], ((1,), (1,))) + bao_ref[...]
    h1 = jax.nn.relu(_dot(attn_out, w1_ref[...], ((1,), (1,))) + b1_ref[...])
    act = _dot(h1, w2_ref[...], ((1,), (1,))) + b2_ref[...]
    x1 = attn_out + act

    x1_ref[...] = x1

    @pl.when(i == 0)
    def _init():
        ssum_ref[...] = jnp.zeros_like(ssum_ref)
        gram_ref[...] = jnp.zeros_like(gram_ref)

    ssum_ref[...] += jnp.sum(x1, axis=0, keepdims=True)
    gram_ref[...] += _dot(x1, x1, ((0,), (0,)))


def _stats_body(ssum_ref, gram_ref, wo_ref, bo_ref, g1_ref, be1_ref,
                g2_ref, be2_ref, at_ref, bfin_ref):
    n = _f32(N2)
    s = ssum_ref[...]
    g = gram_ref[...]
    m1 = s / n
    r = lax.broadcasted_iota(jnp.int32, (C, C), 0)
    c = lax.broadcasted_iota(jnp.int32, (C, C), 1)
    eye = (r == c).astype(_f32)
    diag_g = jnp.sum(g * eye, axis=0, keepdims=True)
    v1 = diag_g / n - m1 * m1
    a1 = g1_ref[...] / jnp.sqrt(v1 + 1e-5)
    c1 = be1_ref[...] - a1 * m1

    wo = wo_ref[...]
    wmod = wo * a1
    bmod = _dot(c1, wo, ((1,), (1,))) + bo_ref[...]
    meany = _dot(m1, wmod, ((1,), (1,))) + bmod
    u = _dot(s, wmod, ((1,), (1,)))
    rmat = _dot(g, wmod, ((1,), (1,)))
    wt = jnp.transpose(wmod)
    qv = jnp.sum(wt * rmat, axis=0, keepdims=True)
    ey2 = qv / n + 2.0 * bmod * u / n + bmod * bmod
    v2 = ey2 - meany * meany
    a2 = g2_ref[...] / jnp.sqrt(v2 + 1e-5)
    c2 = be2_ref[...] - a2 * meany

    at_ref[...] = wt * a2
    bfin_ref[...] = a2 * bmod + c2


def _final_body(x1_ref, at_ref, bfin_ref, out_ref):
    out_ref[...] = jax.nn.relu(
        _dot(x1_ref[...], at_ref[...], ((1,), (0,))) + bfin_ref[...])


def _row(x):
    return x.reshape(1, -1)


def kernel(voxel_features, voxel_coords, query_coords, key_indices, key_mask,
           W_qpos, b_qpos, W_kpos, b_kpos, W_in, b_in, W_ao, b_ao,
           W1, b1, W2, b2, g1, be1, W_o, b_o, g2, be2):
    vcp = jnp.pad(voxel_coords, ((0, 0), (0, CP - 3)))
    qcp = jnp.pad(query_coords, ((0, 0), (0, CP - 3)))
    wkp = jnp.pad(W_kpos, ((0, 0), (0, CP - 3)))
    wqp = jnp.pad(W_qpos, ((0, 0), (0, CP - 3)))
    idx_flat = key_indices.reshape(-1).astype(jnp.int32)
    maskf = key_mask.astype(_f32)

    gf, gc = _sc_gather(voxel_features, vcp, idx_flat)

    full = lambda shape: pl.BlockSpec(shape, lambda i: (0, 0))
    x1, ssum, gram = pl.pallas_call(
        _main_body,
        grid=(NB,),
        in_specs=[
            pl.BlockSpec((BS, C), lambda i: (i, 0)),
            pl.BlockSpec((BS, CP), lambda i: (i, 0)),
            pl.BlockSpec((B, CP), lambda i: (i, 0)),
            pl.BlockSpec((B, S), lambda i: (i, 0)),
            full((C, CP)), full((C, CP)),
            full((3 * C, C)), full((1, 3 * C)),
            full((1, C)), full((1, C)),
            full((C, C)), full((1, C)),
            full((FF, C)), full((1, FF)),
            full((C, FF)), full((1, C)),
        ],
        out_specs=[
            pl.BlockSpec((B, C), lambda i: (i, 0)),
            pl.BlockSpec((1, C), lambda i: (0, 0)),
            pl.BlockSpec((C, C), lambda i: (0, 0)),
        ],
        out_shape=[
            jax.ShapeDtypeStruct((N2, C), _f32),
            jax.ShapeDtypeStruct((1, C), _f32),
            jax.ShapeDtypeStruct((C, C), _f32),
        ],
    )(gf, gc, qcp, maskf, wkp, wqp,
      W_in, _row(b_in), _row(b_kpos), _row(b_qpos), W_ao, _row(b_ao),
      W1, _row(b1), W2, _row(b2))

    at, bfin = pl.pallas_call(
        _stats_body,
        out_shape=[jax.ShapeDtypeStruct((C, C), _f32),
                   jax.ShapeDtypeStruct((1, C), _f32)],
    )(ssum, gram, W_o, _row(b_o), _row(g1), _row(be1), _row(g2), _row(be2))

    out = pl.pallas_call(
        _final_body,
        grid=(NB,),
        in_specs=[
            pl.BlockSpec((B, C), lambda i: (i, 0)),
            full((C, C)),
            full((1, C)),
        ],
        out_specs=pl.BlockSpec((B, C), lambda i: (i, 0)),
        out_shape=jax.ShapeDtypeStruct((N2, C), _f32),
    )(x1, at, bfin)
    return out
